# Initial kernel scaffold; baseline (speedup 1.0000x reference)
import jax
import jax.numpy as jnp
from jax.experimental import pallas as pl


def kernel(word_ids, embedding_weight):
    raise NotImplementedError("write your pallas kernel here")


# speedup vs baseline: 1.1103x; 1.1103x over previous
import functools

import jax
import jax.numpy as jnp
from jax import lax
from jax.experimental import pallas as pl
from jax.experimental.pallas import tpu as pltpu
from jax.experimental.pallas import tpu_sc as plsc

NC = 2
NS = 16
NW = NC * NS

B = 16384 * 50
D = 32
BPW = B // NW
C = 3200
NCHUNK = BPW // C

_mesh = plsc.VectorSubcoreMesh(core_axis_name="c", subcore_axis_name="s")


@functools.partial(
    pl.kernel,
    mesh=_mesh,
    out_type=jax.ShapeDtypeStruct((B, D), jnp.float32),
    compiler_params=pltpu.CompilerParams(use_tc_tiling_on_sc=False),
    scratch_types=[
        pltpu.VMEM((C,), jnp.int32),
        pltpu.VMEM((C, D), jnp.float32),
        pltpu.SemaphoreType.DMA,
    ],
)
def _gather_kernel(idx_hbm, table_hbm, out_hbm, idx_v, rows_v, sem):
    wid = lax.axis_index("s") * NC + lax.axis_index("c")
    base = wid * BPW

    def body(i, carry):
        off = base + i * C
        pltpu.sync_copy(idx_hbm.at[pl.ds(off, C)], idx_v)
        pltpu.async_copy(table_hbm.at[idx_v], rows_v, sem).wait()
        pltpu.sync_copy(rows_v, out_hbm.at[pl.ds(off, C)])
        return carry

    lax.fori_loop(0, NCHUNK, body, 0)


def kernel(word_ids, embedding_weight):
    idx = word_ids.reshape(-1).astype(jnp.int32)
    out = _gather_kernel(idx, embedding_weight)
    return out.reshape(word_ids.shape + (embedding_weight.shape[1],))


# speedup vs baseline: 1.1125x; 1.0020x over previous
import functools

import jax
import jax.numpy as jnp
from jax import lax
from jax.experimental import pallas as pl
from jax.experimental.pallas import tpu as pltpu
from jax.experimental.pallas import tpu_sc as plsc

NC = 2
NS = 16
NW = NC * NS

B = 16384 * 50
D = 32
BPW = B // NW
C = 1600
NCHUNK = BPW // C

_mesh = plsc.VectorSubcoreMesh(core_axis_name="c", subcore_axis_name="s")


@functools.partial(
    pl.kernel,
    mesh=_mesh,
    out_type=jax.ShapeDtypeStruct((B, D), jnp.float32),
    compiler_params=pltpu.CompilerParams(use_tc_tiling_on_sc=False),
    scratch_types=[
        pltpu.VMEM((BPW,), jnp.int32),
        pltpu.VMEM((2, C, D), jnp.float32),
        pltpu.SemaphoreType.DMA,
        pltpu.SemaphoreType.DMA,
        pltpu.SemaphoreType.DMA,
        pltpu.SemaphoreType.DMA,
    ],
)
def _gather_kernel(idx_hbm, table_hbm, out_hbm, idx_v, rows_v, sg0, sg1, so0, so1):
    wid = lax.axis_index("s") * NC + lax.axis_index("c")
    base = wid * BPW
    pltpu.sync_copy(idx_hbm.at[pl.ds(base, BPW)], idx_v)

    sg = (sg0, sg1)
    so = (so0, so1)
    gather_cp = [None] * NCHUNK
    store_cp = [None] * NCHUNK
    for g in range(NCHUNK):
        b = g % 2
        if g >= 2:
            store_cp[g - 2].wait()
        gather_cp[g] = pltpu.async_copy(
            table_hbm.at[idx_v.at[pl.ds(g * C, C)]], rows_v.at[b], sg[b])
        if g >= 1:
            pb = (g - 1) % 2
            gather_cp[g - 1].wait()
            store_cp[g - 1] = pltpu.async_copy(
                rows_v.at[pb], out_hbm.at[pl.ds(base + (g - 1) * C, C)], so[pb])
    gather_cp[NCHUNK - 1].wait()
    lb = (NCHUNK - 1) % 2
    store_cp[NCHUNK - 1] = pltpu.async_copy(
        rows_v.at[lb], out_hbm.at[pl.ds(base + (NCHUNK - 1) * C, C)], so[lb])
    store_cp[NCHUNK - 2].wait()
    store_cp[NCHUNK - 1].wait()


def kernel(word_ids, embedding_weight):
    idx = word_ids.reshape(-1).astype(jnp.int32)
    out = _gather_kernel(idx, embedding_weight)
    return out.reshape(word_ids.shape + (embedding_weight.shape[1],))
